# baseline (speedup 1.0000x reference)
import functools
import jax
import jax.numpy as jnp
import numpy as np
from jax import lax
from jax.experimental import pallas as pl
from jax.experimental.pallas import tpu as pltpu, tpu_sc as plsc

N = 200000
NX, NY = 432, 496
NV = NX * NY
SENT = NV
MAXV = 16000
MAXP = 32
BIG = 1 << 28

NW = 32
CHP = 6272
PCW = CHP * 4
NVP = 221184
RANGE = NVP // NW
JLOC = RANGE // 32
SCSL = NVP // 16
VROWS = MAXV * MAXP
VDUMP = 2048
VTOT = VROWS + VDUMP
RN = 16640
SHARE = 512
ZCH = VTOT * 4 // NW // 16
LW = CHP * 8

_MESH = plsc.VectorSubcoreMesh(core_axis_name="c", subcore_axis_name="s",
                               num_cores=2, num_subcores=16)
_CP = pltpu.CompilerParams(needs_layout_passes=False)
_I32 = jnp.int32
_F32 = jnp.float32


def _iota():
    return lax.iota(_I32, 16)


def _ext(v, k):
    return jnp.sum(jnp.where(_iota() == k, v, 0))


_GDN = lax.GatherDimensionNumbers(
    offset_dims=(), collapsed_slice_dims=(0,), start_index_map=(0,))


def _gat(x, idx):
    return lax.gather(x, idx[:, None], _GDN, (1,),
                      mode=lax.GatherScatterMode.PROMISE_IN_BOUNDS)


def _wid():
    return lax.axis_index("c") * 16 + lax.axis_index("s")


def _flat_of(pcd_v, i, wid):
    it = _iota()
    p = i * 16 + it
    gpos = wid * CHP + p
    valid = gpos < N
    b4 = p * 4
    x = plsc.load_gather(pcd_v, [b4])
    y = plsc.load_gather(pcd_v, [b4 + 1])
    z = plsc.load_gather(pcd_v, [b4 + 2])
    q0 = x / np.float32(0.16)
    q1 = (y - np.float32(-39.68)) / np.float32(0.16)
    q2 = (z - np.float32(-3.0)) / np.float32(4.0)
    inr = ((q0 >= 0) & (q0 < NX) & (q1 >= 0) & (q1 < NY)
           & (q2 >= 0) & (q2 < 1) & valid)
    cx = jnp.where(inr, q0, 0.0).astype(_I32)
    cy = jnp.where(inr, q1, 0.0).astype(_I32)
    return jnp.where(inr, cy * NX + cx, SENT)


@functools.partial(
    pl.kernel, mesh=_MESH, compiler_params=_CP,
    out_type=jax.ShapeDtypeStruct((2 * NVP,), _I32),
    scratch_types=[
        pltpu.VMEM((PCW,), _F32),
        pltpu.VMEM((49, 128), _I32),
        pltpu.VMEM((SCSL,), _I32),
        pltpu.VMEM((128,), _I32),
        pltpu.VMEM_SHARED((NVP,), _I32),
    ],
)
def _k1(pcd_hbm, counts2, pcd_v, flatv, zbuf, ones, csh):
    c = lax.axis_index("c")
    s = lax.axis_index("s")
    wid = c * 16 + s
    it = _iota()
    pltpu.sync_copy(pcd_hbm.at[pl.ds(wid * PCW, PCW)], pcd_v)

    def zb(k, _):
        zbuf[pl.ds(k * 16, 16)] = jnp.zeros((16,), _I32)
        return 0
    lax.fori_loop(0, SCSL // 16, zb, 0)

    def ob(k, _):
        ones[pl.ds(k * 16, 16)] = jnp.ones((16,), _I32)
        return 0
    lax.fori_loop(0, 8, ob, 0)

    def fl(i, _):
        p = i * 16 + it
        flat = _flat_of(pcd_v, i, wid)
        plsc.store_scatter(flatv, [p >> 7, p & 127], flat)
        return 0
    lax.fori_loop(0, CHP // 16, fl, 0)

    pltpu.sync_copy(zbuf, csh.at[pl.ds(s * SCSL, SCSL)])
    plsc.subcore_barrier()

    def hist(k, _):
        pltpu.sync_copy(ones, csh.at[flatv.at[k]], add=True)
        return 0
    lax.fori_loop(0, 49, hist, 0)

    plsc.subcore_barrier()
    pltpu.sync_copy(csh.at[pl.ds(s * SCSL, SCSL)],
                    counts2.at[pl.ds(c * NVP + s * SCSL, SCSL)])


@functools.partial(
    pl.kernel, mesh=_MESH, compiler_params=_CP,
    out_type=(jax.ShapeDtypeStruct((NVP,), _I32),
              jax.ShapeDtypeStruct((512,), _I32)),
    scratch_types=[
        pltpu.VMEM((RANGE,), _I32),
        pltpu.VMEM((RANGE,), _I32),
        pltpu.VMEM((16,), _I32),
    ],
)
def _k2a(counts2, countsM, totals, c0, c1, stage):
    wid = _wid()
    it = _iota()
    base = wid * RANGE
    pltpu.sync_copy(counts2.at[pl.ds(base, RANGE)], c0)
    pltpu.sync_copy(counts2.at[pl.ds(NVP + base, RANGE)], c1)

    def body(k, acc):
        cc = c0[pl.ds(k * 16, 16)] + c1[pl.ds(k * 16, 16)]
        c0[pl.ds(k * 16, 16)] = cc
        v = base + k * 16 + it
        occ = (cc > 0) & (v < NV)
        return acc + occ.astype(_I32)
    acc = lax.fori_loop(0, RANGE // 16, body, jnp.zeros((16,), _I32))
    total = jnp.sum(acc)
    stage[...] = jnp.where(it == 0, total, 0)
    pltpu.sync_copy(c0, countsM.at[pl.ds(base, RANGE)])
    pltpu.sync_copy(stage, totals.at[pl.ds(wid * 16, 16)])


@functools.partial(
    pl.kernel, mesh=_MESH, compiler_params=_CP,
    out_type=jax.ShapeDtypeStruct((32 * RANGE,), _I32),
    scratch_types=[
        pltpu.VMEM((RANGE,), _I32),
        pltpu.VMEM((512,), _I32),
        pltpu.VMEM((RANGE,), _I32),
    ],
)
def _k2b(countsM, totals, slotmapT, cm, tvm, sstage):
    wid = _wid()
    it = _iota()
    base = wid * RANGE

    pltpu.sync_copy(totals, tvm)
    pltpu.sync_copy(countsM.at[pl.ds(base, RANGE)], cm)

    t0 = plsc.load_gather(tvm, [it * 16])
    t1 = plsc.load_gather(tvm, [256 + it * 16])
    cs0 = plsc.cumsum(t0)
    s0 = _ext(cs0, 15)
    cs1 = plsc.cumsum(t1) + s0
    g = wid // 16
    l = wid % 16
    incl = jnp.where(g == 0, _ext(cs0, l), _ext(cs1, l))
    mytot = jnp.where(g == 0, _ext(t0, l), _ext(t1, l))
    sbase = incl - mytot

    def scan(k, slotc):
        cc = cm[pl.ds(k * 16, 16)]
        v = base + k * 16 + it
        occ = (cc > 0) & (v < NV)
        occi = occ.astype(_I32)
        pc = plsc.cumsum(occi)
        slot = slotc + pc - occi
        kept = occ & (slot < MAXV)
        slotval = jnp.where(kept, slot, BIG)
        o = it + 16 * (k & 1)
        plsc.store_scatter(sstage, [o * JLOC + (k >> 1)], slotval)
        return slotc + _ext(pc, 15)

    lax.fori_loop(0, RANGE // 16, scan, sbase)

    for o in range(32):
        pltpu.sync_copy(sstage.at[pl.ds(o * JLOC, JLOC)],
                        slotmapT.at[pl.ds(o * RANGE + wid * JLOC, JLOC)])


@functools.partial(
    pl.kernel, mesh=_MESH, compiler_params=_CP,
    out_type=(jax.ShapeDtypeStruct((32 * LW,), _I32),
              jax.ShapeDtypeStruct((512,), _I32)),
    scratch_types=[
        pltpu.VMEM((PCW,), _F32),
        pltpu.VMEM((CHP,), _I32),
        pltpu.VMEM((CHP,), _I32),
        pltpu.VMEM((CHP,), _I32),
        pltpu.VMEM((LW,), _I32),
        pltpu.VMEM((16,), _I32),
    ],
)
def _k3a(pcd_hbm, slotmap, lists, listcnt, pcd_v, flatv, idxv, slotv, lst, st):
    wid = _wid()
    it = _iota()
    pltpu.sync_copy(pcd_hbm.at[pl.ds(wid * PCW, PCW)], pcd_v)

    def fl(i, _):
        p = i * 16 + it
        flat = _flat_of(pcd_v, i, wid)
        plsc.store_scatter(flatv, [p], flat)
        idx = (flat & 31) * RANGE + (flat >> 5)
        idx = idx + jnp.where(flat == SENT, p & 127, 0)
        plsc.store_scatter(idxv, [p], idx)
        return 0
    lax.fori_loop(0, CHP // 16, fl, 0)

    pltpu.sync_copy(slotmap.at[idxv], slotv)

    def comp(i, fill):
        p = i * 16 + it
        flat = plsc.load_gather(flatv, [p])
        slot = plsc.load_gather(slotv, [p])
        keep = slot < MAXV
        ki = keep.astype(_I32)
        pk = plsc.cumsum(ki)
        posr = fill + pk - 1
        b4 = p * 4
        x = plsc.load_gather(pcd_v, [b4])
        y = plsc.load_gather(pcd_v, [b4 + 1])
        z = plsc.load_gather(pcd_v, [b4 + 2])
        w = plsc.load_gather(pcd_v, [b4 + 3])
        p8 = posr * 8
        plsc.store_scatter(lst, [p8], flat, mask=keep)
        plsc.store_scatter(lst, [p8 + 1], slot, mask=keep)
        plsc.store_scatter(lst, [p8 + 2], plsc.bitcast(x, _I32), mask=keep)
        plsc.store_scatter(lst, [p8 + 3], plsc.bitcast(y, _I32), mask=keep)
        plsc.store_scatter(lst, [p8 + 4], plsc.bitcast(z, _I32), mask=keep)
        plsc.store_scatter(lst, [p8 + 5], plsc.bitcast(w, _I32), mask=keep)
        return fill + _ext(pk, 15)
    fill = lax.fori_loop(0, CHP // 16, comp, jnp.int32(0))

    pltpu.sync_copy(lst, lists.at[pl.ds(wid * LW, LW)])
    st[...] = jnp.where(it == 0, fill, 0)
    pltpu.sync_copy(st, listcnt.at[pl.ds(wid * 16, 16)])


SLOTW = 512
SLABW = SLOTW * 128
NVW = SLOTW * 16


@functools.partial(
    pl.kernel, mesh=_MESH, compiler_params=_CP,
    out_type=(jax.ShapeDtypeStruct((32 * SLABW,), _F32),
              jax.ShapeDtypeStruct((32 * NVW,), _I32)),
    scratch_types=[
        pltpu.VMEM((512,), _I32),
        pltpu.VMEM((SLOTW,), _I32),
        pltpu.VMEM((SLOTW,), _I32),
        pltpu.VMEM((SLOTW,), _I32),
        pltpu.VMEM((8192,), _I32),
        pltpu.VMEM((SLABW,), _F32),
        pltpu.VMEM((NVW,), _I32),
        pltpu.VMEM((16,), _I32),
    ],
)
def _k3b(lists, listcnt, vox, nvh, lc, ctr, vys, vxs, wbuf, slab, nvs, s16):
    o = _wid()
    it = _iota()
    pltpu.sync_copy(listcnt, lc)

    def zs(k, _):
        slab[pl.ds(k * 16, 16)] = jnp.zeros((16,), _F32)
        return 0
    lax.fori_loop(0, SLABW // 16, zs, 0)

    def zc(k, _):
        ctr[pl.ds(k * 16, 16)] = jnp.zeros((16,), _I32)
        return 0
    lax.fori_loop(0, SLOTW // 16, zc, 0)

    kt0 = plsc.load_gather(lc, [it * 16])
    kt1 = plsc.load_gather(lc, [256 + it * 16])
    obase = o * SLOTW

    def tile_loop(t, _):
        kt = jnp.where(t < 16, _ext(kt0, t & 15), _ext(kt1, t & 15))
        nwin = (kt + 1023) // 1024

        def win_loop(w, _):
            pltpu.sync_copy(lists.at[pl.ds(t * LW + w * 8192, 8192)], wbuf)
            nv = jnp.minimum(1024, kt - w * 1024)
            nvr = (nv + 15) // 16

            def vloop(u, _):
                r = u * 16 + it
                gpos = w * 1024 + r
                valid = gpos < kt
                r8 = r * 8
                flat = plsc.load_gather(wbuf, [r8])
                slot_raw = plsc.load_gather(wbuf, [r8 + 1])
                slot = jnp.where(valid, slot_raw, 0x10000 + it)
                own = valid & ((slot >> 9) == o)
                sl = jnp.where(own, slot - obase, 0)
                keys = (slot << 4) | it
                sk, perm = plsc.sort_key_val(keys, it)
                sks = sk >> 4
                prev = _gat(sks, jnp.maximum(it - 1, 0))
                seg = (it == 0) | (sks != prev)
                segn = _gat(seg.astype(_I32), jnp.minimum(it + 1, 15))
                islast_s = (it == 15) | (segn == 1)
                start = plsc.cummax(jnp.where(seg, it, 0))
                pack = (it - start) * 2 + islast_s.astype(_I32)
                plsc.store_scatter(s16, [perm], pack)
                pk16 = s16[...]
                r16 = pk16 >> 1
                islast = (pk16 & 1) == 1
                cnt = plsc.load_gather(ctr, [sl])
                rank = cnt + r16
                plsc.store_scatter(ctr, [sl], rank + 1, mask=own & islast)
                kept = own & (rank < MAXP)
                off = jnp.where(kept, sl * 128 + rank * 4, 0)
                for q in range(4):
                    xq = plsc.bitcast(plsc.load_gather(wbuf, [r8 + 2 + q]),
                                      _F32)
                    plsc.store_scatter(slab, [off + q], xq, mask=kept)
                m0 = own & (rank == 0)
                vy = flat // NX
                vx = flat - vy * NX
                plsc.store_scatter(vys, [sl], vy, mask=m0)
                plsc.store_scatter(vxs, [sl], vx, mask=m0)
                return 0
            return lax.fori_loop(0, nvr, vloop, 0)
        return lax.fori_loop(0, nwin, win_loop, 0)

    lax.fori_loop(0, 32, tile_loop, 0)

    def nvrow(k, _):
        sl16 = k * 16 + it
        c = ctr[pl.ds(k * 16, 16)]
        vy = vys[pl.ds(k * 16, 16)]
        vx = vxs[pl.ds(k * 16, 16)]
        has = c > 0
        p16 = sl16 * 16
        plsc.store_scatter(nvs, [p16], jnp.where(has, jnp.minimum(c, MAXP), 0))
        plsc.store_scatter(nvs, [p16 + 1], jnp.where(has, 0, -1))
        plsc.store_scatter(nvs, [p16 + 2], jnp.where(has, vy, -1))
        plsc.store_scatter(nvs, [p16 + 3], jnp.where(has, vx, -1))
        return 0
    lax.fori_loop(0, SLOTW // 16, nvrow, 0)

    pltpu.sync_copy(slab, vox.at[pl.ds(o * SLABW, SLABW)])
    pltpu.sync_copy(nvs, nvh.at[pl.ds(o * NVW, NVW)])


def kernel(pcd):
    pcdf = jnp.concatenate(
        [pcd.reshape(-1), jnp.zeros((NW * PCW - 4 * N,), _F32)])
    counts2 = _k1(pcdf)
    countsM, totals = _k2a(counts2)
    slotmapT = _k2b(countsM, totals)
    lists, listcnt = _k3a(pcdf, slotmapT)
    vox, nvh = _k3b(lists, listcnt)
    voxels = vox.reshape(32 * SLOTW, MAXP, 4)[:MAXV]
    nvb = nvh.reshape(32 * SLOTW, 16)
    num_points = nvb[:MAXV, 0]
    vcoords = nvb[:MAXV, 1:4]
    return voxels, vcoords, num_points


# speedup vs baseline: 1.0304x; 1.0304x over previous
import functools
import jax
import jax.numpy as jnp
import numpy as np
from jax import lax
from jax.experimental import pallas as pl
from jax.experimental.pallas import tpu as pltpu, tpu_sc as plsc

N = 200000
NX, NY = 432, 496
NV = NX * NY
SENT = NV
MAXV = 16000
MAXP = 32
BIG = 1 << 28

NW = 32
CHP = 6272
PCW = CHP * 4
NVP = 221184
RANGE = NVP // NW
JLOC = RANGE // 32
SCSL = NVP // 16
VROWS = MAXV * MAXP
VDUMP = 2048
VTOT = VROWS + VDUMP
RN = 16640
SHARE = 512
ZCH = VTOT * 4 // NW // 16
LW = CHP * 8

_MESH = plsc.VectorSubcoreMesh(core_axis_name="c", subcore_axis_name="s",
                               num_cores=2, num_subcores=16)
_CP = pltpu.CompilerParams(needs_layout_passes=False)
_I32 = jnp.int32
_F32 = jnp.float32


def _iota():
    return lax.iota(_I32, 16)


def _ext(v, k):
    return jnp.sum(jnp.where(_iota() == k, v, 0))


_GDN = lax.GatherDimensionNumbers(
    offset_dims=(), collapsed_slice_dims=(0,), start_index_map=(0,))


def _gat(x, idx):
    return lax.gather(x, idx[:, None], _GDN, (1,),
                      mode=lax.GatherScatterMode.PROMISE_IN_BOUNDS)


def _wid():
    return lax.axis_index("c") * 16 + lax.axis_index("s")


def _flat_of(pcd_v, i, wid):
    it = _iota()
    p = i * 16 + it
    gpos = wid * CHP + p
    valid = gpos < N
    b4 = p * 4
    x = plsc.load_gather(pcd_v, [b4])
    y = plsc.load_gather(pcd_v, [b4 + 1])
    z = plsc.load_gather(pcd_v, [b4 + 2])
    q0 = x / np.float32(0.16)
    q1 = (y - np.float32(-39.68)) / np.float32(0.16)
    q2 = (z - np.float32(-3.0)) / np.float32(4.0)
    inr = ((q0 >= 0) & (q0 < NX) & (q1 >= 0) & (q1 < NY)
           & (q2 >= 0) & (q2 < 1) & valid)
    cx = jnp.where(inr, q0, 0.0).astype(_I32)
    cy = jnp.where(inr, q1, 0.0).astype(_I32)
    return jnp.where(inr, cy * NX + cx, SENT)


@functools.partial(
    pl.kernel, mesh=_MESH, compiler_params=_CP,
    out_type=jax.ShapeDtypeStruct((2 * NVP,), _I32),
    scratch_types=[
        pltpu.VMEM((PCW,), _F32),
        pltpu.VMEM((49, 128), _I32),
        pltpu.VMEM((SCSL,), _I32),
        pltpu.VMEM((128,), _I32),
        pltpu.VMEM_SHARED((NVP,), _I32),
    ],
)
def _k1(pcd_hbm, counts2, pcd_v, flatv, zbuf, ones, csh):
    c = lax.axis_index("c")
    s = lax.axis_index("s")
    wid = c * 16 + s
    it = _iota()
    pltpu.sync_copy(pcd_hbm.at[pl.ds(wid * PCW, PCW)], pcd_v)

    def zb(k, _):
        zbuf[pl.ds(k * 16, 16)] = jnp.zeros((16,), _I32)
        return 0
    lax.fori_loop(0, SCSL // 16, zb, 0)

    def ob(k, _):
        ones[pl.ds(k * 16, 16)] = jnp.ones((16,), _I32)
        return 0
    lax.fori_loop(0, 8, ob, 0)

    def fl(i, _):
        p = i * 16 + it
        flat = _flat_of(pcd_v, i, wid)
        plsc.store_scatter(flatv, [p >> 7, p & 127], flat)
        return 0
    lax.fori_loop(0, CHP // 16, fl, 0)

    pltpu.sync_copy(zbuf, csh.at[pl.ds(s * SCSL, SCSL)])
    plsc.subcore_barrier()

    def hist(k, _):
        pltpu.sync_copy(ones, csh.at[flatv.at[k]], add=True)
        return 0
    lax.fori_loop(0, 49, hist, 0)

    plsc.subcore_barrier()
    pltpu.sync_copy(csh.at[pl.ds(s * SCSL, SCSL)],
                    counts2.at[pl.ds(c * NVP + s * SCSL, SCSL)])


@functools.partial(
    pl.kernel, mesh=_MESH, compiler_params=_CP,
    out_type=(jax.ShapeDtypeStruct((NVP,), _I32),
              jax.ShapeDtypeStruct((512,), _I32)),
    scratch_types=[
        pltpu.VMEM((RANGE,), _I32),
        pltpu.VMEM((RANGE,), _I32),
        pltpu.VMEM((16,), _I32),
    ],
)
def _k2a(counts2, countsM, totals, c0, c1, stage):
    wid = _wid()
    it = _iota()
    base = wid * RANGE
    pltpu.sync_copy(counts2.at[pl.ds(base, RANGE)], c0)
    pltpu.sync_copy(counts2.at[pl.ds(NVP + base, RANGE)], c1)

    def body(k, acc):
        cc = c0[pl.ds(k * 16, 16)] + c1[pl.ds(k * 16, 16)]
        c0[pl.ds(k * 16, 16)] = cc
        v = base + k * 16 + it
        occ = (cc > 0) & (v < NV)
        return acc + occ.astype(_I32)
    acc = lax.fori_loop(0, RANGE // 16, body, jnp.zeros((16,), _I32))
    total = jnp.sum(acc)
    stage[...] = jnp.where(it == 0, total, 0)
    pltpu.sync_copy(c0, countsM.at[pl.ds(base, RANGE)])
    pltpu.sync_copy(stage, totals.at[pl.ds(wid * 16, 16)])


@functools.partial(
    pl.kernel, mesh=_MESH, compiler_params=_CP,
    out_type=jax.ShapeDtypeStruct((32 * RANGE,), _I32),
    scratch_types=[
        pltpu.VMEM((RANGE,), _I32),
        pltpu.VMEM((512,), _I32),
        pltpu.VMEM((RANGE,), _I32),
    ],
)
def _k2b(countsM, totals, slotmapT, cm, tvm, sstage):
    wid = _wid()
    it = _iota()
    base = wid * RANGE

    pltpu.sync_copy(totals, tvm)
    pltpu.sync_copy(countsM.at[pl.ds(base, RANGE)], cm)

    t0 = plsc.load_gather(tvm, [it * 16])
    t1 = plsc.load_gather(tvm, [256 + it * 16])
    cs0 = plsc.cumsum(t0)
    s0 = _ext(cs0, 15)
    cs1 = plsc.cumsum(t1) + s0
    g = wid // 16
    l = wid % 16
    incl = jnp.where(g == 0, _ext(cs0, l), _ext(cs1, l))
    mytot = jnp.where(g == 0, _ext(t0, l), _ext(t1, l))
    sbase = incl - mytot

    def scan(k, slotc):
        cc = cm[pl.ds(k * 16, 16)]
        v = base + k * 16 + it
        occ = (cc > 0) & (v < NV)
        occi = occ.astype(_I32)
        pc = plsc.cumsum(occi)
        slot = slotc + pc - occi
        kept = occ & (slot < MAXV)
        slotval = jnp.where(kept, slot, BIG)
        o = it + 16 * (k & 1)
        plsc.store_scatter(sstage, [o * JLOC + (k >> 1)], slotval)
        return slotc + _ext(pc, 15)

    lax.fori_loop(0, RANGE // 16, scan, sbase)

    for o in range(32):
        pltpu.sync_copy(sstage.at[pl.ds(o * JLOC, JLOC)],
                        slotmapT.at[pl.ds(o * RANGE + wid * JLOC, JLOC)])


@functools.partial(
    pl.kernel, mesh=_MESH, compiler_params=_CP,
    out_type=(jax.ShapeDtypeStruct((32 * LW,), _I32),
              jax.ShapeDtypeStruct((512,), _I32)),
    scratch_types=[
        pltpu.VMEM((PCW,), _F32),
        pltpu.VMEM((CHP,), _I32),
        pltpu.VMEM((CHP,), _I32),
        pltpu.VMEM((CHP,), _I32),
        pltpu.VMEM((LW,), _I32),
        pltpu.VMEM((16,), _I32),
    ],
)
def _k3a(pcd_hbm, slotmap, lists, listcnt, pcd_v, flatv, idxv, slotv, lst, st):
    wid = _wid()
    it = _iota()
    pltpu.sync_copy(pcd_hbm.at[pl.ds(wid * PCW, PCW)], pcd_v)

    def fl(i, _):
        p = i * 16 + it
        flat = _flat_of(pcd_v, i, wid)
        plsc.store_scatter(flatv, [p], flat)
        idx = (flat & 31) * RANGE + (flat >> 5)
        idx = idx + jnp.where(flat == SENT, p & 127, 0)
        plsc.store_scatter(idxv, [p], idx)
        return 0
    lax.fori_loop(0, CHP // 16, fl, 0)

    pltpu.sync_copy(slotmap.at[idxv], slotv)

    def comp(i, fill):
        p = i * 16 + it
        flat = plsc.load_gather(flatv, [p])
        slot = plsc.load_gather(slotv, [p])
        keep = slot < MAXV
        ki = keep.astype(_I32)
        pk = plsc.cumsum(ki)
        posr = fill + pk - 1
        b4 = p * 4
        x = plsc.load_gather(pcd_v, [b4])
        y = plsc.load_gather(pcd_v, [b4 + 1])
        z = plsc.load_gather(pcd_v, [b4 + 2])
        w = plsc.load_gather(pcd_v, [b4 + 3])
        p8 = posr * 8
        plsc.store_scatter(lst, [p8], flat, mask=keep)
        plsc.store_scatter(lst, [p8 + 1], slot, mask=keep)
        plsc.store_scatter(lst, [p8 + 2], plsc.bitcast(x, _I32), mask=keep)
        plsc.store_scatter(lst, [p8 + 3], plsc.bitcast(y, _I32), mask=keep)
        plsc.store_scatter(lst, [p8 + 4], plsc.bitcast(z, _I32), mask=keep)
        plsc.store_scatter(lst, [p8 + 5], plsc.bitcast(w, _I32), mask=keep)
        return fill + _ext(pk, 15)
    fill = lax.fori_loop(0, CHP // 16, comp, jnp.int32(0))

    pltpu.sync_copy(lst, lists.at[pl.ds(wid * LW, LW)])
    st[...] = jnp.where(it == 0, fill, 0)
    pltpu.sync_copy(st, listcnt.at[pl.ds(wid * 16, 16)])


SLOTW = 512
SLABW = SLOTW * 128
LASTW = MAXV - 31 * SLOTW


@functools.partial(
    pl.kernel, mesh=_MESH, compiler_params=_CP,
    out_type=(jax.ShapeDtypeStruct((MAXV * 128,), _F32),
              jax.ShapeDtypeStruct((MAXV,), _I32),
              jax.ShapeDtypeStruct((MAXV * 3,), _I32)),
    scratch_types=[
        pltpu.VMEM((512,), _I32),
        pltpu.VMEM((SLOTW,), _I32),
        pltpu.VMEM((SLOTW,), _I32),
        pltpu.VMEM((SLOTW,), _I32),
        pltpu.VMEM((8192,), _I32),
        pltpu.VMEM((SLABW,), _F32),
        pltpu.VMEM((SLOTW,), _I32),
        pltpu.VMEM((SLOTW * 3,), _I32),
        pltpu.VMEM((16,), _I32),
    ],
)
def _k3b(lists, listcnt, vox, nph, vch, lc, ctr, vys, vxs, wbuf, slab,
         nps, vcs, s16):
    o = _wid()
    it = _iota()
    pltpu.sync_copy(listcnt, lc)

    def zs(k, _):
        slab[pl.ds(k * 16, 16)] = jnp.zeros((16,), _F32)
        return 0
    lax.fori_loop(0, SLABW // 16, zs, 0)

    def zc(k, _):
        ctr[pl.ds(k * 16, 16)] = jnp.zeros((16,), _I32)
        return 0
    lax.fori_loop(0, SLOTW // 16, zc, 0)

    kt0 = plsc.load_gather(lc, [it * 16])
    kt1 = plsc.load_gather(lc, [256 + it * 16])
    obase = o * SLOTW

    def tile_loop(t, _):
        kt = jnp.where(t < 16, _ext(kt0, t & 15), _ext(kt1, t & 15))
        nwin = (kt + 1023) // 1024

        def win_loop(w, _):
            pltpu.sync_copy(lists.at[pl.ds(t * LW + w * 8192, 8192)], wbuf)
            nv = jnp.minimum(1024, kt - w * 1024)
            nvr = (nv + 15) // 16

            def vloop(u, _):
                r = u * 16 + it
                gpos = w * 1024 + r
                valid = gpos < kt
                r8 = r * 8
                flat = plsc.load_gather(wbuf, [r8])
                slot_raw = plsc.load_gather(wbuf, [r8 + 1])
                slot = jnp.where(valid, slot_raw, 0x10000 + it)
                own = valid & ((slot >> 9) == o)
                sl = jnp.where(own, slot - obase, 0)
                keys = (slot << 4) | it
                sk, perm = plsc.sort_key_val(keys, it)
                sks = sk >> 4
                prev = _gat(sks, jnp.maximum(it - 1, 0))
                seg = (it == 0) | (sks != prev)
                segn = _gat(seg.astype(_I32), jnp.minimum(it + 1, 15))
                islast_s = (it == 15) | (segn == 1)
                start = plsc.cummax(jnp.where(seg, it, 0))
                pack = (it - start) * 2 + islast_s.astype(_I32)
                plsc.store_scatter(s16, [perm], pack)
                pk16 = s16[...]
                r16 = pk16 >> 1
                islast = (pk16 & 1) == 1
                cnt = plsc.load_gather(ctr, [sl])
                rank = cnt + r16
                plsc.store_scatter(ctr, [sl], rank + 1, mask=own & islast)
                kept = own & (rank < MAXP)
                off = jnp.where(kept, sl * 128 + rank * 4, 0)
                for q in range(4):
                    xq = plsc.bitcast(plsc.load_gather(wbuf, [r8 + 2 + q]),
                                      _F32)
                    plsc.store_scatter(slab, [off + q], xq, mask=kept)
                m0 = own & (rank == 0)
                vy = flat // NX
                vx = flat - vy * NX
                plsc.store_scatter(vys, [sl], vy, mask=m0)
                plsc.store_scatter(vxs, [sl], vx, mask=m0)
                return 0
            return lax.fori_loop(0, nvr, vloop, 0)
        return lax.fori_loop(0, nwin, win_loop, 0)

    lax.fori_loop(0, 32, tile_loop, 0)

    def nvrow(k, _):
        sl16 = k * 16 + it
        c = ctr[pl.ds(k * 16, 16)]
        vy = vys[pl.ds(k * 16, 16)]
        vx = vxs[pl.ds(k * 16, 16)]
        has = c > 0
        nps[pl.ds(k * 16, 16)] = jnp.where(has, jnp.minimum(c, MAXP), 0)
        p3 = sl16 * 3
        plsc.store_scatter(vcs, [p3], jnp.where(has, 0, -1))
        plsc.store_scatter(vcs, [p3 + 1], jnp.where(has, vy, -1))
        plsc.store_scatter(vcs, [p3 + 2], jnp.where(has, vx, -1))
        return 0
    lax.fori_loop(0, SLOTW // 16, nvrow, 0)

    def full_out():
        pltpu.sync_copy(slab, vox.at[pl.ds(o * SLABW, SLABW)])
        pltpu.sync_copy(nps, nph.at[pl.ds(o * SLOTW, SLOTW)])
        pltpu.sync_copy(vcs, vch.at[pl.ds(o * SLOTW * 3, SLOTW * 3)])
    pl.when(o < 31)(full_out)

    def last_out():
        pltpu.sync_copy(slab.at[pl.ds(0, LASTW * 128)],
                        vox.at[pl.ds(31 * SLABW, LASTW * 128)])
        pltpu.sync_copy(nps.at[pl.ds(0, LASTW)],
                        nph.at[pl.ds(31 * SLOTW, LASTW)])
        pltpu.sync_copy(vcs.at[pl.ds(0, LASTW * 3)],
                        vch.at[pl.ds(31 * SLOTW * 3, LASTW * 3)])
    pl.when(o == 31)(last_out)


def kernel(pcd):
    pcdf = jnp.concatenate(
        [pcd.reshape(-1), jnp.zeros((NW * PCW - 4 * N,), _F32)])
    counts2 = _k1(pcdf)
    countsM, totals = _k2a(counts2)
    slotmapT = _k2b(countsM, totals)
    lists, listcnt = _k3a(pcdf, slotmapT)
    vox, nph, vch = _k3b(lists, listcnt)
    voxels = vox.reshape(MAXV, MAXP, 4)
    vcoords = vch.reshape(MAXV, 3)
    return voxels, vcoords, nph
